# Initial kernel scaffold; baseline (speedup 1.0000x reference)
import jax
import jax.numpy as jnp
from jax.experimental import pallas as pl


def kernel(xs, wt_pad):
    raise NotImplementedError("write your pallas kernel here")


# speedup vs baseline: 3.1141x; 3.1141x over previous
import jax
import jax.numpy as jnp
from jax.experimental import pallas as pl
from jax.experimental.pallas import tpu as pltpu

_SUBLANE = 8
_TILE_THRESHOLD = 1024


def _round_up(n, m):
    return ((n + m - 1) // m) * m


def _body(in_dim):
    def body(x_ref, w_ref, o_ref):
        o_ref[...] = jax.lax.dot_general(
            x_ref[...],
            w_ref[0:in_dim, :],
            dimension_numbers=(((1,), (0,)), ((), ())),
            preferred_element_type=jnp.float32,
        ).astype(o_ref.dtype)

    return body


def _tiled(x, wt_pad, rows_pad, tile_rows):
    rows, in_dim = x.shape
    in_pad, out_pad = wt_pad.shape
    if rows_pad != rows:
        x = jnp.pad(x, ((0, rows_pad - rows), (0, 0)))
    grid = (rows_pad // tile_rows,)
    return pl.pallas_call(
        _body(in_dim),
        out_shape=jax.ShapeDtypeStruct((rows_pad, out_pad), x.dtype),
        grid=grid,
        in_specs=[
            pl.BlockSpec((tile_rows, in_dim), lambda i: (i, 0)),
            pl.BlockSpec((in_pad, out_pad), lambda i: (0, 0)),
        ],
        out_specs=pl.BlockSpec((tile_rows, out_pad), lambda i: (i, 0)),
        compiler_params=pltpu.CompilerParams(
            dimension_semantics=("parallel",)
        ),
        cost_estimate=pl.CostEstimate(
            flops=2 * rows * in_dim * out_pad,
            transcendentals=0,
            bytes_accessed=4 * (rows * in_dim + in_pad * out_pad
                                + rows_pad * out_pad),
        ),
    )(x, wt_pad)


@jax.jit
def kernel(xs, wt_pad):
    n, b, in_dim = xs.shape
    rows = n * b
    x = xs.reshape(rows, in_dim)
    if rows < _TILE_THRESHOLD:
        rows_pad = _round_up(max(rows, _SUBLANE), _SUBLANE)
        return _tiled(x, wt_pad, rows_pad, rows_pad)
    rows_pad = _round_up(rows, 512)
    tile_rows = 512
    for cand in (4096, 2048, 1024):
        if rows_pad % cand == 0:
            tile_rows = cand
            break
    return _tiled(x, wt_pad, rows_pad, tile_rows)
